# Initial kernel scaffold; baseline (speedup 1.0000x reference)
import jax
import jax.numpy as jnp
from jax.experimental import pallas as pl


def kernel(query, key, value):
    raise NotImplementedError("write your pallas kernel here")


# speedup vs baseline: 1.8967x; 1.8967x over previous
import functools

import jax
import jax.numpy as jnp
from jax import lax
from jax.experimental import pallas as pl
from jax.experimental.pallas import tpu as pltpu
from jax.experimental.pallas import tpu_sc as plsc

_B, _C, _N = 4, 768, 4096
_GOFF = (0, 96, 192, 384)
_GSZ = (96, 96, 192, 384)
_CHUNK = 8
_NCHUNK = 96 // _CHUNK


_TILE = 128
_NTILE = _C // _TILE


def _row_sum_tree(a):
    ncols = a.shape[1]
    acc = a[:, 0:128]
    for t in range(1, ncols // 128):
        acc = acc + a[:, t * 128:(t + 1) * 128]
    g = acc[:, 0:8]
    for i in range(1, 16):
        g = g + acc[:, i * 8:(i + 1) * 8]
    g = g[:, 0:4] + g[:, 4:8]
    g = g[:, 0:2] + g[:, 2:4]
    return g[:, 0:1] + g[:, 1:2]


def _center_body(q_ref, x_ref):
    q4 = q_ref[...]
    cf = (((q4[0] + q4[1]) + q4[2]) + q4[3]) * jnp.float32(0.25)
    m = _row_sum_tree(cf) / jnp.float32(_N)
    x_ref[...] = cf - m


def _similarity_body(xf_ref, xft_ref, ms_ref):
    fact = jnp.float32(_N - 1)
    g = lax.dot_general(xf_ref[...], xft_ref[...], (((1,), (0,)), ((), ())),
                        preferred_element_type=jnp.float32)
    cov = g / fact
    ii = lax.broadcasted_iota(jnp.int32, (_C, _C), 0)
    jj = lax.broadcasted_iota(jnp.int32, (_C, _C), 1)
    d = jnp.sum(jnp.where(ii == jj, cov, 0.0), axis=1)
    s = jnp.sqrt(d)
    corr = cov * pl.reciprocal(s[:, None] * s[None, :])
    corr = jnp.clip(corr, -1.0, 1.0)
    ms_ref[0, :] = (_row_sum_tree(corr) / jnp.float32(_C))[:, 0]


def _rank_body(msf_ref, mst_ref, rank_ref):
    t = pl.program_id(0)
    msj = msf_ref[0, :]
    msi = mst_ref[0, 0, :]
    ii = t * _TILE + lax.broadcasted_iota(jnp.int32, (_TILE, _C), 0)
    jj = lax.broadcasted_iota(jnp.int32, (_TILE, _C), 1)
    ahead = (msj[None, :] > msi[:, None]) | \
        ((msj[None, :] == msi[:, None]) & (jj < ii))
    rank_ref[0, 0, :] = jnp.sum(ahead.astype(jnp.int32), axis=1)


def _invert_body(rankf_ref, sidx_ref):
    t = pl.program_id(0)
    rank = rankf_ref[0, :]
    kk = t * _TILE + lax.broadcasted_iota(jnp.int32, (_TILE, _C), 0)
    jj = lax.broadcasted_iota(jnp.int32, (_TILE, _C), 1)
    eq = rank[None, :] == kk
    sidx_ref[0, 0, :] = jnp.sum(jnp.where(eq, jj, 0), axis=1)


def _sorted_channel_ids(query, interpret=False):
    x = pl.pallas_call(
        _center_body,
        grid=(_NTILE,),
        in_specs=[pl.BlockSpec((_B, _TILE, _N), lambda t: (0, t, 0))],
        out_specs=pl.BlockSpec((_TILE, _N), lambda t: (t, 0)),
        out_shape=jax.ShapeDtypeStruct((_C, _N), jnp.float32),
        interpret=interpret,
    )(query)
    ms = pl.pallas_call(
        _similarity_body,
        in_specs=[
            pl.BlockSpec((_C, _N), lambda: (0, 0)),
            pl.BlockSpec((_N, _C), lambda: (0, 0)),
        ],
        out_specs=pl.BlockSpec((1, _C), lambda: (0, 0)),
        out_shape=jax.ShapeDtypeStruct((1, _C), jnp.float32),
        interpret=interpret,
    )(x, x.T)
    ms3 = ms.reshape(_NTILE, 1, _TILE)
    rank = pl.pallas_call(
        _rank_body,
        grid=(_NTILE,),
        in_specs=[
            pl.BlockSpec((1, _C), lambda t: (0, 0)),
            pl.BlockSpec((1, 1, _TILE), lambda t: (t, 0, 0)),
        ],
        out_specs=pl.BlockSpec((1, 1, _TILE), lambda t: (t, 0, 0)),
        out_shape=jax.ShapeDtypeStruct((_NTILE, 1, _TILE), jnp.int32),
        interpret=interpret,
    )(ms, ms3)
    return pl.pallas_call(
        _invert_body,
        grid=(_NTILE,),
        in_specs=[pl.BlockSpec((1, _C), lambda t: (0, 0))],
        out_specs=pl.BlockSpec((1, 1, _TILE), lambda t: (t, 0, 0)),
        out_shape=jax.ShapeDtypeStruct((_NTILE, 1, _TILE), jnp.int32),
        interpret=interpret,
    )(rank.reshape(1, _C))


def _regroup_body(q_hbm, k_hbm, v_hbm, sidx_hbm, *refs):
    outs = tuple(tuple(refs[t * 4 + g] for g in range(4)) for t in range(3))
    idx_v, buf0, buf1, sem0, sem1 = refs[12:]
    srcs = (q_hbm, k_hbm, v_hbm)
    bufs = (buf0, buf1)
    sems = (sem0, sem1)

    wid = lax.axis_index("s") * 2 + lax.axis_index("c")
    b = wid // 8
    slot = wid % 8
    j0 = slot * 96
    pltpu.sync_copy(sidx_hbm.at[pl.ds(j0, 96)], idx_v)
    boff = (b * _C).astype(jnp.int32)
    for i in range(6):
        sl = pl.ds(i * 16, 16)
        idx_v[sl] = idx_v[sl] + boff

    def run_group(g, in_group):
        off, sz = _GOFF[g], _GSZ[g]

        @pl.when(in_group)
        def _():
            dst0 = b * sz + (j0 - off)
            for t in range(3):
                src = srcs[t]
                out = outs[t][g]

                def gather(c, buf_i):
                    return pltpu.make_async_copy(
                        src.at[idx_v.at[pl.ds(c * _CHUNK, _CHUNK)]],
                        bufs[buf_i], sems[buf_i])

                gather(0, 0).start()

                def pair(i, _, src=src, out=out, dst0=dst0):
                    for u in range(2):
                        c = i * 2 + u
                        nxt = (u + 1) % 2

                        @pl.when(c + 1 < _NCHUNK)
                        def _():
                            gather(c + 1, nxt).start()

                        gather(c, u).wait()
                        pltpu.sync_copy(
                            bufs[u], out.at[pl.ds(dst0 + c * _CHUNK, _CHUNK)])
                    return 0

                lax.fori_loop(0, _NCHUNK // 2, pair, 0)

    run_group(0, slot == 0)
    run_group(1, slot == 1)
    run_group(2, jnp.logical_and(slot >= 2, slot < 4))
    run_group(3, slot >= 4)


def _regroup(q2, k2, v2, sidx):
    mesh = plsc.VectorSubcoreMesh(core_axis_name="c", subcore_axis_name="s")
    out_type = tuple(
        jax.ShapeDtypeStruct((_B * sz, _N), jnp.float32)
        for _ in range(3) for sz in _GSZ)
    fn = pl.kernel(
        _regroup_body,
        out_type=out_type,
        mesh=mesh,
        scratch_types=[
            pltpu.VMEM((96,), jnp.int32),
            pltpu.VMEM((_CHUNK, _N), jnp.float32),
            pltpu.VMEM((_CHUNK, _N), jnp.float32),
            pltpu.SemaphoreType.DMA,
            pltpu.SemaphoreType.DMA,
        ],
    )
    return fn(q2, k2, v2, sidx)


def kernel(query, key, value):
    sidx = _sorted_channel_ids(query).reshape(_C)
    q2 = query.reshape(_B * _C, _N)
    k2 = key.reshape(_B * _C, _N)
    v2 = value.reshape(_B * _C, _N)
    flat = _regroup(q2, k2, v2, sidx)
    groups = tuple(
        tuple(flat[t * 4 + g].reshape(_B, _GSZ[g], _N) for g in range(4))
        for t in range(3))
    return groups
